# Initial kernel scaffold; baseline (speedup 1.0000x reference)
import jax
import jax.numpy as jnp
from jax import lax
from jax.experimental import pallas as pl
from jax.experimental.pallas import tpu as pltpu
from jax.experimental.pallas import tpu_sc as plsc

_VOCAB = 1000
_EMB = 64
_BATCH = 16384
_HIST = 50

_NC = 2
_NS = 16
_NW = _NC * _NS

_B = _BATCH * _HIST
_PER_W = _B // _NW
_IDXW = 128
_G = 4
_C = _IDXW * _G
_NCHUNKS = _PER_W // _C
_NROWS_IDX = _PER_W // _IDXW


def _body(ids_hbm, table_hbm, out_hbm, idx_v, buf0, buf1,
          gsem0, gsem1, ssem0, ssem1):
    c_id = lax.axis_index("c")
    s_id = lax.axis_index("s")
    wid = s_id * _NC + c_id
    base = wid * _PER_W

    pltpu.sync_copy(ids_hbm.at[wid], idx_v)

    bufs = (buf0, buf1)
    gsems = (gsem0, gsem1)
    ssems = (ssem0, ssem1)

    def gather_descs(c, b):
        return [
            pltpu.make_async_copy(
                table_hbm.at[idx_v.at[c * _G + j]],
                bufs[b].at[pl.ds(j * _IDXW, _IDXW)],
                gsems[b],
            )
            for j in range(_G)
        ]

    def store_desc(c, b):
        return pltpu.make_async_copy(
            bufs[b], out_hbm.at[pl.ds(base + c * _C, _C)], ssems[b]
        )

    def fire_gathers(c, b):
        for d in gather_descs(c, b):
            d.start()

    fire_gathers(0, 0)
    fire_gathers(1, 1)

    def loop_body(i, carry):
        for b in range(2):
            c = 2 * i + b
            for d in gather_descs(c, b):
                d.wait()
            store_desc(c, b).start()
            store_desc(c, b).wait()

            @pl.when(c + 2 < _NCHUNKS)
            def _():
                fire_gathers(c + 2, b)
        return carry

    lax.fori_loop(0, _NCHUNKS // 2, loop_body, 0)


def kernel(ids, table):
    ids_r = ids.reshape(_NW, _NROWS_IDX, _IDXW)
    run = pl.kernel(
        _body,
        out_type=jax.ShapeDtypeStruct((_B, _EMB), jnp.float32),
        mesh=plsc.VectorSubcoreMesh(core_axis_name="c", subcore_axis_name="s"),
        scratch_types=[
            pltpu.VMEM((_NROWS_IDX, _IDXW), jnp.int32),
            pltpu.VMEM((_C, _EMB), jnp.float32),
            pltpu.VMEM((_C, _EMB), jnp.float32),
            pltpu.SemaphoreType.DMA,
            pltpu.SemaphoreType.DMA,
            pltpu.SemaphoreType.DMA,
            pltpu.SemaphoreType.DMA,
        ],
    )
    out = run(ids_r, table)
    return out.reshape(_BATCH, _HIST, _EMB)


# speedup vs baseline: 5.2645x; 5.2645x over previous
import jax
import jax.numpy as jnp
from jax import lax
from jax.experimental import pallas as pl
from jax.experimental.pallas import tpu as pltpu
from jax.experimental.pallas import tpu_sc as plsc

_VOCAB = 1000
_EMB = 64
_BATCH = 16384
_HIST = 50

_NC = 2
_NS = 16
_NW = _NC * _NS

_B = _BATCH * _HIST
_PER_W = _B // _NW
_IDXW = 128
_G = 4
_C = _IDXW * _G
_NCHUNKS = _PER_W // _C
_NROWS_IDX = _PER_W // _IDXW


def _body(ids_hbm, table_hbm, out_hbm, idx_v, buf0, buf1,
          gsem0, gsem1, ssem0, ssem1):
    c_id = lax.axis_index("c")
    s_id = lax.axis_index("s")
    wid = s_id * _NC + c_id
    base = wid * _PER_W

    pltpu.sync_copy(ids_hbm.at[wid], idx_v)

    bufs = (buf0, buf1)
    gsems = (gsem0, gsem1)
    ssems = (ssem0, ssem1)

    def gather_descs(c, b):
        return [
            pltpu.make_async_copy(
                table_hbm.at[idx_v.at[c * _G + j]],
                bufs[b].at[pl.ds(j * _IDXW, _IDXW)],
                gsems[b],
            )
            for j in range(_G)
        ]

    def store_desc(c, b):
        return pltpu.make_async_copy(
            bufs[b], out_hbm.at[pl.ds(base + c * _C, _C)], ssems[b]
        )

    def fire_gathers(c, b):
        for d in gather_descs(c, b):
            d.start()

    fire_gathers(0, 0)
    fire_gathers(1, 1)

    def loop_body(i, carry):
        for b in range(2):
            c = 2 * i + b
            for d in gather_descs(c, b):
                d.wait()
            store_desc(c, b).start()
            store_desc(c, b).wait()

            @pl.when(c + 2 < _NCHUNKS)
            def _():
                fire_gathers(c + 2, b)
        return carry

    lax.fori_loop(0, _NCHUNKS // 2, loop_body, 0)


def kernel(ids, table):
    ids_r = ids.reshape(_NW, _NROWS_IDX, _IDXW)
    run = pl.kernel(
        _body,
        out_type=jax.ShapeDtypeStruct((_B, _EMB), jnp.float32),
        mesh=plsc.VectorSubcoreMesh(core_axis_name="c", subcore_axis_name="s"),
        compiler_params=pltpu.CompilerParams(use_tc_tiling_on_sc=False),
        scratch_types=[
            pltpu.VMEM((_NROWS_IDX, _IDXW), jnp.int32),
            pltpu.VMEM((_C, _EMB), jnp.float32),
            pltpu.VMEM((_C, _EMB), jnp.float32),
            pltpu.SemaphoreType.DMA,
            pltpu.SemaphoreType.DMA,
            pltpu.SemaphoreType.DMA,
            pltpu.SemaphoreType.DMA,
        ],
    )
    out = run(ids_r, table)
    return out.reshape(_BATCH, _HIST, _EMB)


# speedup vs baseline: 5.2793x; 1.0028x over previous
import jax
import jax.numpy as jnp
from jax import lax
from jax.experimental import pallas as pl
from jax.experimental.pallas import tpu as pltpu
from jax.experimental.pallas import tpu_sc as plsc

_VOCAB = 1000
_EMB = 64
_BATCH = 16384
_HIST = 50

_NC = 2
_NS = 16
_NW = _NC * _NS

_B = _BATCH * _HIST
_PER_W = _B // _NW
_IDXW = 128
_G = 2
_C = _IDXW * _G
_NCHUNKS = _PER_W // _C
_NBUF = 4
_NROWS_IDX = _PER_W // _IDXW


def _body(ids_hbm, table_hbm, out_hbm, idx_v, buf0, buf1, buf2, buf3,
          gsem0, gsem1, gsem2, gsem3, ssem0, ssem1, ssem2, ssem3):
    c_id = lax.axis_index("c")
    s_id = lax.axis_index("s")
    wid = s_id * _NC + c_id
    base = wid * _PER_W

    pltpu.sync_copy(ids_hbm.at[wid], idx_v)

    bufs = (buf0, buf1, buf2, buf3)
    gsems = (gsem0, gsem1, gsem2, gsem3)
    ssems = (ssem0, ssem1, ssem2, ssem3)

    def gather_descs(c, b):
        return [
            pltpu.make_async_copy(
                table_hbm.at[idx_v.at[c * _G + j]],
                bufs[b].at[pl.ds(j * _IDXW, _IDXW)],
                gsems[b],
            )
            for j in range(_G)
        ]

    def store_desc(c, b):
        return pltpu.make_async_copy(
            bufs[b], out_hbm.at[pl.ds(base + c * _C, _C)], ssems[b]
        )

    def fire_gathers(c, b):
        for d in gather_descs(c, b):
            d.start()

    fire_gathers(0, 0)

    def loop_body(i, carry):
        for b in range(_NBUF):
            c = 4 * i + b
            nb = (b + 1) % _NBUF

            @pl.when(c >= _NBUF - 1)
            def _():
                store_desc(c - (_NBUF - 1), nb).wait()

            @pl.when(c + 1 < _NCHUNKS)
            def _():
                fire_gathers(c + 1, nb)

            for d in gather_descs(c, b):
                d.wait()
            store_desc(c, b).start()
        return carry

    lax.fori_loop(0, _NCHUNKS // _NBUF, loop_body, 0)

    for c in range(_NCHUNKS - (_NBUF - 1), _NCHUNKS):
        store_desc(c, c % _NBUF).wait()


def kernel(ids, table):
    ids_r = ids.reshape(_NW, _NROWS_IDX, _IDXW)
    run = pl.kernel(
        _body,
        out_type=jax.ShapeDtypeStruct((_B, _EMB), jnp.float32),
        mesh=plsc.VectorSubcoreMesh(core_axis_name="c", subcore_axis_name="s"),
        compiler_params=pltpu.CompilerParams(use_tc_tiling_on_sc=False),
        scratch_types=[
            pltpu.VMEM((_NROWS_IDX, _IDXW), jnp.int32),
            pltpu.VMEM((_C, _EMB), jnp.float32),
            pltpu.VMEM((_C, _EMB), jnp.float32),
            pltpu.VMEM((_C, _EMB), jnp.float32),
            pltpu.VMEM((_C, _EMB), jnp.float32),
            pltpu.SemaphoreType.DMA,
            pltpu.SemaphoreType.DMA,
            pltpu.SemaphoreType.DMA,
            pltpu.SemaphoreType.DMA,
            pltpu.SemaphoreType.DMA,
            pltpu.SemaphoreType.DMA,
            pltpu.SemaphoreType.DMA,
            pltpu.SemaphoreType.DMA,
        ],
    )
    out = run(ids_r, table)
    return out.reshape(_BATCH, _HIST, _EMB)
